# baseline (speedup 1.0000x reference)
import functools

import jax
import jax.numpy as jnp
from jax import lax
from jax.experimental import pallas as pl
from jax.experimental.pallas import tpu as pltpu
from jax.experimental.pallas import tpu_sc as plsc

_L = 16
_NC = 2
_NS = 16
_NW = _NC * _NS


def _tc_project(h, Wr, Wc):
    N, H = h.shape
    BN = 1000

    def body(h_ref, wr_ref, wc_ref, pr_ref, pc_ref):
        hb = h_ref[...]
        pr_ref[...] = jnp.dot(hb, wr_ref[...], preferred_element_type=jnp.float32)
        pc_ref[...] = jnp.dot(hb, wc_ref[...], preferred_element_type=jnp.float32)

    return pl.pallas_call(
        body,
        grid=(N // BN,),
        in_specs=[
            pl.BlockSpec((BN, H), lambda i: (i, 0)),
            pl.BlockSpec((H, H), lambda i: (0, 0)),
            pl.BlockSpec((H, H), lambda i: (0, 0)),
        ],
        out_specs=[
            pl.BlockSpec((BN, H), lambda i: (i, 0)),
            pl.BlockSpec((BN, H), lambda i: (i, 0)),
        ],
        out_shape=[jax.ShapeDtypeStruct((N, H), jnp.float32)] * 2,
    )(h, Wr, Wc)


def _sc_gather(Prow, Pcol, px, py, pz, row, col):
    N, H = Prow.shape
    E = row.shape[0]
    EPW = E // _NW
    CH = 80
    NCHUNK = EPW // CH
    GR = CH // _L
    NB = 3

    mesh = plsc.VectorSubcoreMesh(
        core_axis_name="c", subcore_axis_name="s",
        num_cores=_NC, num_subcores=_NS)

    @functools.partial(
        pl.kernel,
        out_type=[
            jax.ShapeDtypeStruct((E, H), jnp.float32),
            jax.ShapeDtypeStruct((E,), jnp.float32),
        ],
        mesh=mesh,
        compiler_params=pltpu.CompilerParams(needs_layout_passes=False),
        scratch_types=[
            pltpu.VMEM((N,), jnp.float32),
            pltpu.VMEM((N,), jnp.float32),
            pltpu.VMEM((N,), jnp.float32),
            pltpu.VMEM((EPW,), jnp.int32),
            pltpu.VMEM((EPW,), jnp.int32),
            pltpu.VMEM((EPW,), jnp.float32),
            [pltpu.VMEM((CH, H), jnp.float32) for _ in range(NB)],
            [pltpu.VMEM((CH, H), jnp.float32) for _ in range(NB)],
            [pltpu.SemaphoreType.DMA for _ in range(NB)],
            [pltpu.SemaphoreType.DMA for _ in range(NB)],
        ],
    )
    def k(prow_hbm, pcol_hbm, px_hbm, py_hbm, pz_hbm, row_hbm, col_hbm,
          z_hbm, d2_hbm,
          px_v, py_v, pz_v, idx_r, idx_c, d2_all, bufA, bufB, semg, semz):
        cid = lax.axis_index("c")
        sid = lax.axis_index("s")
        wid = cid * _NS + sid
        base0 = wid * EPW
        pltpu.sync_copy(row_hbm.at[pl.ds(base0, EPW)], idx_r)
        pltpu.sync_copy(col_hbm.at[pl.ds(base0, EPW)], idx_c)
        pltpu.sync_copy(px_hbm, px_v)
        pltpu.sync_copy(py_hbm, py_v)
        pltpu.sync_copy(pz_hbm, pz_v)

        def issue(j, s):
            pltpu.async_copy(
                prow_hbm.at[idx_r.at[pl.ds(j * CH, CH)]], bufA[s], semg[s])
            pltpu.async_copy(
                pcol_hbm.at[idx_c.at[pl.ds(j * CH, CH)]], bufB[s], semg[s])

        def wait_z(s):
            pltpu.make_async_copy(
                bufA[s], z_hbm.at[pl.ds(base0, CH)], semz[s]).wait()

        def process(j, s):
            base = base0 + j * CH
            for g in range(GR):
                sl = pl.ds(j * CH + g * _L, _L)
                rv = idx_r[sl]
                cv = idx_c[sl]
                ex = plsc.load_gather(px_v, [rv]) - plsc.load_gather(px_v, [cv]) + 1e-8
                ey = plsc.load_gather(py_v, [rv]) - plsc.load_gather(py_v, [cv]) + 1e-8
                ez = plsc.load_gather(pz_v, [rv]) - plsc.load_gather(pz_v, [cv]) + 1e-8
                d2_all[sl] = ex * ex + ey * ey + ez * ez

            pltpu.make_async_copy(prow_hbm.at[idx_r.at[pl.ds(j * CH, CH)]],
                                  bufA[s], semg[s]).wait()
            pltpu.make_async_copy(pcol_hbm.at[idx_c.at[pl.ds(j * CH, CH)]],
                                  bufB[s], semg[s]).wait()

            def addrow(r, c2):
                for q in range(H // _L):
                    qsl = pl.ds(q * _L, _L)
                    bufA[s][r, qsl] = bufA[s][r, qsl] + bufB[s][r, qsl]
                return c2
            lax.fori_loop(0, CH, addrow, 0)

            pltpu.async_copy(bufA[s], z_hbm.at[pl.ds(base, CH)], semz[s])

        issue(0, 0)
        issue(1, 1)

        def body3(i, carry):
            j = 3 * i
            for t in range(3):
                jj = j + t
                jn = jj + 2
                s_next = (t + 2) % 3

                @pl.when(jn < NCHUNK)
                def _():
                    @pl.when(jn >= NB)
                    def _():
                        wait_z(s_next)
                    issue(jn, s_next)

                @pl.when(jj < NCHUNK)
                def _():
                    process(jj, t % 3)
            return carry

        lax.fori_loop(0, (NCHUNK + 2) // 3, body3, 0)

        for s in range(NB):
            wait_z(s)
        pltpu.sync_copy(d2_all, d2_hbm.at[pl.ds(base0, EPW)])

    return k(Prow, Pcol, px, py, pz, row, col)


def _tc_edge(z, d2, wd, be1, We2, be2, ge, bge, Wx1, bx1, Wx2, bx2):
    E, H = z.shape
    BE = 8000

    def body(z_ref, d2_ref, wd_ref, be1_ref, w2_ref,
             be2_ref, ge_ref, bge_ref, wx1_ref, bx1_ref, wx2_ref, bx2_ref,
             em_ref, c_ref):
        dist = jnp.sqrt(d2_ref[...])
        x = z_ref[...] + dist * wd_ref[...] + be1_ref[...]
        x = x * jax.nn.sigmoid(x)
        m = jnp.dot(x.astype(jnp.bfloat16), w2_ref[...],
                    preferred_element_type=jnp.float32) + be2_ref[...]
        mu = jnp.mean(m, axis=-1, keepdims=True)
        var = jnp.mean((m - mu) * (m - mu), axis=-1, keepdims=True)
        em = (m - mu) / jnp.sqrt(var + 1e-5) * ge_ref[...] + bge_ref[...]
        em_ref[...] = em
        t = jnp.dot(em.astype(jnp.bfloat16), wx1_ref[...],
                    preferred_element_type=jnp.float32) + bx1_ref[...]
        t = t * jax.nn.sigmoid(t)
        c_ref[...] = jnp.tanh(
            jnp.dot(t, wx2_ref[...], preferred_element_type=jnp.float32) + bx2_ref[...])

    full = lambda i: (0, 0)
    col1 = lambda i: (i, 0)
    return pl.pallas_call(
        body,
        grid=(E // BE,),
        in_specs=[
            pl.BlockSpec((BE, H), col1),
            pl.BlockSpec((BE, 1), col1),
            pl.BlockSpec((1, H), full),
            pl.BlockSpec((1, H), full),
            pl.BlockSpec((H, H), full),
            pl.BlockSpec((1, H), full),
            pl.BlockSpec((1, H), full),
            pl.BlockSpec((1, H), full),
            pl.BlockSpec((H, H), full),
            pl.BlockSpec((1, H), full),
            pl.BlockSpec((H, 1), full),
            pl.BlockSpec((1, 1), full),
        ],
        out_specs=[
            pl.BlockSpec((BE, H), col1),
            pl.BlockSpec((BE, 1), col1),
        ],
        out_shape=[
            jax.ShapeDtypeStruct((E, H), jnp.float32),
            jax.ShapeDtypeStruct((E, 1), jnp.float32),
        ],
    )(z, d2, wd, be1, We2, be2, ge, bge, Wx1, bx1, Wx2, bx2)


def _sc_scatter_em(em, col, zem):
    E, H = em.shape
    N = zem.shape[0]
    EPW = E // _NW
    CH = 80
    NCHUNK = EPW // CH
    GR = CH // _L

    mesh = plsc.VectorSubcoreMesh(
        core_axis_name="c", subcore_axis_name="s",
        num_cores=_NC, num_subcores=_NS)

    @functools.partial(
        pl.kernel,
        out_type=[jax.ShapeDtypeStruct((_NC, N, H), jnp.float32)],
        mesh=mesh,
        compiler_params=pltpu.CompilerParams(needs_layout_passes=False),
        scratch_types=[
            pltpu.VMEM((EPW,), jnp.int32),
            pltpu.VMEM((CH,), jnp.int32),
            [pltpu.VMEM((CH, H), jnp.float32) for _ in range(2)],
            [pltpu.SemaphoreType.DMA for _ in range(2)],
            pltpu.VMEM_SHARED((N, H), jnp.float32),
        ],
    )
    def k(em_hbm, col_hbm, zem_hbm, hmp_hbm,
          idx_c, idx_chunk, em_buf, seme, acc_em):
        cid = lax.axis_index("c")
        sid = lax.axis_index("s")
        wid = cid * _NS + sid
        base0 = wid * EPW

        @pl.when(sid == 0)
        def _():
            pltpu.sync_copy(zem_hbm, acc_em)

        pltpu.sync_copy(col_hbm.at[pl.ds(base0, EPW)], idx_c)

        plsc.subcore_barrier()

        def issue(j, s):
            pltpu.async_copy(
                em_hbm.at[pl.ds(base0 + j * CH, CH)], em_buf[s], seme[s])

        def process(j, s):
            for g in range(GR):
                idx_chunk[pl.ds(g * _L, _L)] = idx_c[pl.ds(j * CH + g * _L, _L)]
            pltpu.make_async_copy(
                em_hbm.at[pl.ds(base0 + j * CH, CH)], em_buf[s], seme[s]).wait()
            pltpu.sync_copy(em_buf[s], acc_em.at[idx_chunk], add=True)

        issue(0, 0)

        def body2(i, carry):
            j0 = 2 * i
            j1 = j0 + 1

            @pl.when(j1 < NCHUNK)
            def _():
                issue(j1, 1)
            process(j0, 0)

            @pl.when(j0 + 2 < NCHUNK)
            def _():
                issue(j0 + 2, 0)

            @pl.when(j1 < NCHUNK)
            def _():
                process(j1, 1)
            return carry

        lax.fori_loop(0, (NCHUNK + 1) // 2, body2, 0)

        plsc.subcore_barrier()

        @pl.when(sid == 0)
        def _():
            pltpu.sync_copy(acc_em, hmp_hbm.at[cid])

    return k(em, col, zem)


def _sc_scatter_pos(cvals, px, py, pz, row, col, zv):
    E = cvals.shape[0]
    N = zv.shape[0]
    EPW = E // _NW
    NG = EPW // _L

    mesh = plsc.VectorSubcoreMesh(
        core_axis_name="c", subcore_axis_name="s",
        num_cores=_NC, num_subcores=_NS)

    @functools.partial(
        pl.kernel,
        out_type=[
            jax.ShapeDtypeStruct((_NW * N,), jnp.float32),
            jax.ShapeDtypeStruct((_NW * N,), jnp.float32),
            jax.ShapeDtypeStruct((_NW * N,), jnp.float32),
        ],
        mesh=mesh,
        compiler_params=pltpu.CompilerParams(needs_layout_passes=False),
        scratch_types=[
            pltpu.VMEM((N,), jnp.float32),
            pltpu.VMEM((N,), jnp.float32),
            pltpu.VMEM((N,), jnp.float32),
            pltpu.VMEM((EPW,), jnp.int32),
            pltpu.VMEM((EPW,), jnp.int32),
            pltpu.VMEM((EPW,), jnp.float32),
            pltpu.VMEM((N,), jnp.float32),
            pltpu.VMEM((N,), jnp.float32),
            pltpu.VMEM((N,), jnp.float32),
        ],
    )
    def k(c_hbm, px_hbm, py_hbm, pz_hbm, row_hbm, col_hbm, zv_hbm,
          wpx_hbm, wpy_hbm, wpz_hbm,
          px_v, py_v, pz_v, idx_r, idx_c, c_all, awx, awy, awz):
        cid = lax.axis_index("c")
        sid = lax.axis_index("s")
        wid = cid * _NS + sid
        base0 = wid * EPW

        pltpu.sync_copy(row_hbm.at[pl.ds(base0, EPW)], idx_r)
        pltpu.sync_copy(col_hbm.at[pl.ds(base0, EPW)], idx_c)
        pltpu.sync_copy(c_hbm.at[pl.ds(base0, EPW)], c_all)
        pltpu.sync_copy(px_hbm, px_v)
        pltpu.sync_copy(py_hbm, py_v)
        pltpu.sync_copy(pz_hbm, pz_v)
        pltpu.sync_copy(zv_hbm, awx)
        pltpu.sync_copy(zv_hbm, awy)
        pltpu.sync_copy(zv_hbm, awz)

        def group(g, carry):
            sl = pl.ds(g * _L, _L)
            rv = idx_r[sl]
            cv = idx_c[sl]
            cc = c_all[sl]
            wx = (plsc.load_gather(px_v, [rv]) - plsc.load_gather(px_v, [cv])) * cc
            wy = (plsc.load_gather(py_v, [rv]) - plsc.load_gather(py_v, [cv])) * cc
            wz = (plsc.load_gather(pz_v, [rv]) - plsc.load_gather(pz_v, [cv])) * cc
            plsc.addupdate_scatter(awx, [cv], wx)
            plsc.addupdate_scatter(awy, [cv], wy)
            plsc.addupdate_scatter(awz, [cv], wz)
            return carry

        lax.fori_loop(0, NG, group, 0)

        pltpu.sync_copy(awx, wpx_hbm.at[pl.ds(wid * N, N)])
        pltpu.sync_copy(awy, wpy_hbm.at[pl.ds(wid * N, N)])
        pltpu.sync_copy(awz, wpz_hbm.at[pl.ds(wid * N, N)])

    return k(cvals, px, py, pz, row, col, zv)


def _tc_node(h, hmp, wpx, wpy, wpz, Wh1, bh1, Wh2, bh2, gh, bgh):
    N, H = h.shape
    BN = 1000
    W = wpx.shape[0]
    full = lambda i: (0, 0)
    col1 = lambda i: (i, 0)

    def body(h_ref, hmp_ref, wpx_ref, wpy_ref, wpz_ref, w1_ref, b1_ref,
             w2_ref, b2_ref, g_ref, bg_ref, hu_ref, px_ref, py_ref, pz_ref):
        s = h_ref[...] + hmp_ref[0] + hmp_ref[1]
        x = jnp.dot(s, w1_ref[...], preferred_element_type=jnp.float32) + b1_ref[...]
        x = x * jax.nn.sigmoid(x)
        u = jnp.dot(x, w2_ref[...], preferred_element_type=jnp.float32) + b2_ref[...]
        mu = jnp.mean(u, axis=-1, keepdims=True)
        var = jnp.mean((u - mu) * (u - mu), axis=-1, keepdims=True)
        hu_ref[...] = (u - mu) / jnp.sqrt(var + 1e-5) * g_ref[...] + bg_ref[...]
        px_ref[...] = jnp.sum(wpx_ref[...], axis=0, keepdims=True)
        py_ref[...] = jnp.sum(wpy_ref[...], axis=0, keepdims=True)
        pz_ref[...] = jnp.sum(wpz_ref[...], axis=0, keepdims=True)

    return pl.pallas_call(
        body,
        grid=(N // BN,),
        in_specs=[
            pl.BlockSpec((BN, H), col1),
            pl.BlockSpec((_NC, BN, H), lambda i: (0, i, 0)),
            pl.BlockSpec((W, N), full),
            pl.BlockSpec((W, N), full),
            pl.BlockSpec((W, N), full),
            pl.BlockSpec((H, H), full),
            pl.BlockSpec((1, H), full),
            pl.BlockSpec((H, H), full),
            pl.BlockSpec((1, H), full),
            pl.BlockSpec((1, H), full),
            pl.BlockSpec((1, H), full),
        ],
        out_specs=[
            pl.BlockSpec((BN, H), col1),
            pl.BlockSpec((1, N), full),
            pl.BlockSpec((1, N), full),
            pl.BlockSpec((1, N), full),
        ],
        out_shape=[
            jax.ShapeDtypeStruct((N, H), jnp.float32),
            jax.ShapeDtypeStruct((1, N), jnp.float32),
            jax.ShapeDtypeStruct((1, N), jnp.float32),
            jax.ShapeDtypeStruct((1, N), jnp.float32),
        ],
    )(h, hmp, wpx, wpy, wpz, Wh1, bh1, Wh2, bh2, gh, bgh)


def kernel(h, pos, edge_index, edge_attr, We1, be1, We2, be2, ge, bge,
           Wh1, bh1, Wh2, bh2, gh, bgh, Wx1, bx1, Wx2, bx2):
    N, H = h.shape
    E = edge_index.shape[1]

    row = jnp.clip(edge_index[0], 0, N - 1)
    col = jnp.clip(edge_index[1], 0, N - 1)
    px = pos[:, 0]
    py = pos[:, 1]
    pz = pos[:, 2]

    Wr = We1[:H]
    Wc = We1[H:2 * H]
    wd = We1[2 * H].reshape(1, H)

    Prow, Pcol = _tc_project(h, Wr, Wc)
    z, d2 = _sc_gather(Prow, Pcol, px, py, pz, row, col)
    em, c = _tc_edge(
        z, d2.reshape(E, 1), wd, be1.reshape(1, H),
        We2.astype(jnp.bfloat16), be2.reshape(1, H),
        ge.reshape(1, H), bge.reshape(1, H), Wx1.astype(jnp.bfloat16),
        bx1.reshape(1, H), Wx2, bx2.reshape(1, 1))
    zem = jnp.zeros((N, H), jnp.float32)
    zv = jnp.zeros((N,), jnp.float32)
    (hmp,) = _sc_scatter_em(em, col, zem)
    wpx, wpy, wpz = _sc_scatter_pos(c.reshape(E), px, py, pz, row, col, zv)
    hu, pux, puy, puz = _tc_node(
        h, hmp, wpx.reshape(_NW, N), wpy.reshape(_NW, N), wpz.reshape(_NW, N),
        Wh1, bh1.reshape(1, H), Wh2, bh2.reshape(1, H),
        gh.reshape(1, H), bgh.reshape(1, H))
    pos_updates = jnp.concatenate(
        [pux.reshape(N, 1), puy.reshape(N, 1), puz.reshape(N, 1)], axis=1)
    return hu, pos_updates


# speedup vs baseline: 1.0154x; 1.0154x over previous
import functools

import jax
import jax.numpy as jnp
from jax import lax
from jax.experimental import pallas as pl
from jax.experimental.pallas import tpu as pltpu
from jax.experimental.pallas import tpu_sc as plsc

_L = 16
_NC = 2
_NS = 16
_NW = _NC * _NS


def _tc_project(h, Wr, Wc):
    N, H = h.shape
    BN = 1000

    def body(h_ref, wr_ref, wc_ref, pr_ref, pc_ref):
        hb = h_ref[...]
        pr_ref[...] = jnp.dot(hb, wr_ref[...], preferred_element_type=jnp.float32)
        pc_ref[...] = jnp.dot(hb, wc_ref[...], preferred_element_type=jnp.float32)

    return pl.pallas_call(
        body,
        grid=(N // BN,),
        in_specs=[
            pl.BlockSpec((BN, H), lambda i: (i, 0)),
            pl.BlockSpec((H, H), lambda i: (0, 0)),
            pl.BlockSpec((H, H), lambda i: (0, 0)),
        ],
        out_specs=[
            pl.BlockSpec((BN, H), lambda i: (i, 0)),
            pl.BlockSpec((BN, H), lambda i: (i, 0)),
        ],
        out_shape=[jax.ShapeDtypeStruct((N, H), jnp.float32)] * 2,
    )(h, Wr, Wc)


def _sc_gather(Prow, Pcol, px, py, pz, row, col):
    N, H = Prow.shape
    E = row.shape[0]
    EPW = E // _NW
    CH = 80
    NCHUNK = EPW // CH
    GR = CH // _L
    NB = 3

    mesh = plsc.VectorSubcoreMesh(
        core_axis_name="c", subcore_axis_name="s",
        num_cores=_NC, num_subcores=_NS)

    @functools.partial(
        pl.kernel,
        out_type=[
            jax.ShapeDtypeStruct((E, H), jnp.float32),
            jax.ShapeDtypeStruct((E,), jnp.float32),
        ],
        mesh=mesh,
        compiler_params=pltpu.CompilerParams(needs_layout_passes=False),
        scratch_types=[
            pltpu.VMEM((N,), jnp.float32),
            pltpu.VMEM((N,), jnp.float32),
            pltpu.VMEM((N,), jnp.float32),
            pltpu.VMEM((EPW,), jnp.int32),
            pltpu.VMEM((EPW,), jnp.int32),
            pltpu.VMEM((EPW,), jnp.float32),
            [pltpu.VMEM((CH, H), jnp.float32) for _ in range(NB)],
            [pltpu.VMEM((CH, H), jnp.float32) for _ in range(NB)],
            [pltpu.SemaphoreType.DMA for _ in range(NB)],
            [pltpu.SemaphoreType.DMA for _ in range(NB)],
        ],
    )
    def k(prow_hbm, pcol_hbm, px_hbm, py_hbm, pz_hbm, row_hbm, col_hbm,
          z_hbm, d2_hbm,
          px_v, py_v, pz_v, idx_r, idx_c, d2_all, bufA, bufB, semg, semz):
        cid = lax.axis_index("c")
        sid = lax.axis_index("s")
        wid = cid * _NS + sid
        base0 = wid * EPW
        pltpu.sync_copy(row_hbm.at[pl.ds(base0, EPW)], idx_r)
        pltpu.sync_copy(col_hbm.at[pl.ds(base0, EPW)], idx_c)
        pltpu.sync_copy(px_hbm, px_v)
        pltpu.sync_copy(py_hbm, py_v)
        pltpu.sync_copy(pz_hbm, pz_v)

        def issue(j, s):
            pltpu.async_copy(
                prow_hbm.at[idx_r.at[pl.ds(j * CH, CH)]], bufA[s], semg[s])
            pltpu.async_copy(
                pcol_hbm.at[idx_c.at[pl.ds(j * CH, CH)]], bufB[s], semg[s])

        def wait_z(s):
            pltpu.make_async_copy(
                bufA[s], z_hbm.at[pl.ds(base0, CH)], semz[s]).wait()

        def process(j, s):
            base = base0 + j * CH
            for g in range(GR):
                sl = pl.ds(j * CH + g * _L, _L)
                rv = idx_r[sl]
                cv = idx_c[sl]
                ex = plsc.load_gather(px_v, [rv]) - plsc.load_gather(px_v, [cv]) + 1e-8
                ey = plsc.load_gather(py_v, [rv]) - plsc.load_gather(py_v, [cv]) + 1e-8
                ez = plsc.load_gather(pz_v, [rv]) - plsc.load_gather(pz_v, [cv]) + 1e-8
                d2_all[sl] = ex * ex + ey * ey + ez * ez

            pltpu.make_async_copy(prow_hbm.at[idx_r.at[pl.ds(j * CH, CH)]],
                                  bufA[s], semg[s]).wait()
            pltpu.make_async_copy(pcol_hbm.at[idx_c.at[pl.ds(j * CH, CH)]],
                                  bufB[s], semg[s]).wait()

            def addrow(r, c2):
                for q in range(H // _L):
                    qsl = pl.ds(q * _L, _L)
                    bufA[s][r, qsl] = bufA[s][r, qsl] + bufB[s][r, qsl]
                return c2
            lax.fori_loop(0, CH, addrow, 0)

            pltpu.async_copy(bufA[s], z_hbm.at[pl.ds(base, CH)], semz[s])

        issue(0, 0)
        issue(1, 1)

        def body3(i, carry):
            j = 3 * i
            for t in range(3):
                jj = j + t
                jn = jj + 2
                s_next = (t + 2) % 3

                @pl.when(jn < NCHUNK)
                def _():
                    @pl.when(jn >= NB)
                    def _():
                        wait_z(s_next)
                    issue(jn, s_next)

                @pl.when(jj < NCHUNK)
                def _():
                    process(jj, t % 3)
            return carry

        lax.fori_loop(0, (NCHUNK + 2) // 3, body3, 0)

        for s in range(NB):
            wait_z(s)
        pltpu.sync_copy(d2_all, d2_hbm.at[pl.ds(base0, EPW)])

    return k(Prow, Pcol, px, py, pz, row, col)


def _tc_edge(z, d2, wd, be1, We2, be2, ge, bge, Wx1, bx1, Wx2, bx2):
    E, H = z.shape
    BE = 8000
    assert E % BE == 0

    def body(z_ref, d2_ref, wd_ref, be1_ref, w2_ref,
             be2_ref, ge_ref, bge_ref, wx1_ref, bx1_ref, wx2_ref, bx2_ref,
             em_ref, c_ref):
        dist = jnp.sqrt(d2_ref[...])
        x = z_ref[...] + dist * wd_ref[...] + be1_ref[...]
        x = x * jax.nn.sigmoid(x)
        m = jnp.dot(x.astype(jnp.bfloat16), w2_ref[...],
                    preferred_element_type=jnp.float32) + be2_ref[...]
        mu = jnp.mean(m, axis=-1, keepdims=True)
        var = jnp.mean((m - mu) * (m - mu), axis=-1, keepdims=True)
        em = (m - mu) / jnp.sqrt(var + 1e-5) * ge_ref[...] + bge_ref[...]
        em_ref[...] = em
        t = jnp.dot(em.astype(jnp.bfloat16), wx1_ref[...],
                    preferred_element_type=jnp.float32) + bx1_ref[...]
        t = t * jax.nn.sigmoid(t)
        c_ref[...] = jnp.tanh(
            jnp.dot(t, wx2_ref[...], preferred_element_type=jnp.float32) + bx2_ref[...])

    full = lambda i: (0, 0)
    col1 = lambda i: (i, 0)
    return pl.pallas_call(
        body,
        grid=(E // BE,),
        in_specs=[
            pl.BlockSpec((BE, H), col1),
            pl.BlockSpec((BE, 1), col1),
            pl.BlockSpec((1, H), full),
            pl.BlockSpec((1, H), full),
            pl.BlockSpec((H, H), full),
            pl.BlockSpec((1, H), full),
            pl.BlockSpec((1, H), full),
            pl.BlockSpec((1, H), full),
            pl.BlockSpec((H, H), full),
            pl.BlockSpec((1, H), full),
            pl.BlockSpec((H, 1), full),
            pl.BlockSpec((1, 1), full),
        ],
        out_specs=[
            pl.BlockSpec((BE, H), col1),
            pl.BlockSpec((BE, 1), col1),
        ],
        out_shape=[
            jax.ShapeDtypeStruct((E, H), jnp.float32),
            jax.ShapeDtypeStruct((E, 1), jnp.float32),
        ],
    )(z, d2, wd, be1, We2, be2, ge, bge, Wx1, bx1, Wx2, bx2)


def _sc_scatter_em(em, col, zem):
    E, H = em.shape
    N = zem.shape[0]
    EPW = E // _NW
    CH = 80
    NCHUNK = EPW // CH
    GR = CH // _L

    mesh = plsc.VectorSubcoreMesh(
        core_axis_name="c", subcore_axis_name="s",
        num_cores=_NC, num_subcores=_NS)

    @functools.partial(
        pl.kernel,
        out_type=[jax.ShapeDtypeStruct((_NC, N, H), jnp.float32)],
        mesh=mesh,
        compiler_params=pltpu.CompilerParams(needs_layout_passes=False),
        scratch_types=[
            pltpu.VMEM((EPW,), jnp.int32),
            pltpu.VMEM((CH,), jnp.int32),
            [pltpu.VMEM((CH, H), jnp.float32) for _ in range(2)],
            [pltpu.SemaphoreType.DMA for _ in range(2)],
            pltpu.VMEM_SHARED((N, H), jnp.float32),
        ],
    )
    def k(em_hbm, col_hbm, zem_hbm, hmp_hbm,
          idx_c, idx_chunk, em_buf, seme, acc_em):
        cid = lax.axis_index("c")
        sid = lax.axis_index("s")
        wid = cid * _NS + sid
        base0 = wid * EPW

        @pl.when(sid == 0)
        def _():
            pltpu.sync_copy(zem_hbm, acc_em)

        pltpu.sync_copy(col_hbm.at[pl.ds(base0, EPW)], idx_c)

        plsc.subcore_barrier()

        def issue(j, s):
            pltpu.async_copy(
                em_hbm.at[pl.ds(base0 + j * CH, CH)], em_buf[s], seme[s])

        def process(j, s):
            for g in range(GR):
                idx_chunk[pl.ds(g * _L, _L)] = idx_c[pl.ds(j * CH + g * _L, _L)]
            pltpu.make_async_copy(
                em_hbm.at[pl.ds(base0 + j * CH, CH)], em_buf[s], seme[s]).wait()
            pltpu.sync_copy(em_buf[s], acc_em.at[idx_chunk], add=True)

        issue(0, 0)

        def body2(i, carry):
            j0 = 2 * i
            j1 = j0 + 1

            @pl.when(j1 < NCHUNK)
            def _():
                issue(j1, 1)
            process(j0, 0)

            @pl.when(j0 + 2 < NCHUNK)
            def _():
                issue(j0 + 2, 0)

            @pl.when(j1 < NCHUNK)
            def _():
                process(j1, 1)
            return carry

        lax.fori_loop(0, (NCHUNK + 1) // 2, body2, 0)

        plsc.subcore_barrier()

        @pl.when(sid == 0)
        def _():
            pltpu.sync_copy(acc_em, hmp_hbm.at[cid])

    return k(em, col, zem)


def _sc_scatter_pos(cvals, px, py, pz, row, col, zv):
    E = cvals.shape[0]
    N = zv.shape[0]
    EPW = E // _NW
    NG = EPW // _L

    mesh = plsc.VectorSubcoreMesh(
        core_axis_name="c", subcore_axis_name="s",
        num_cores=_NC, num_subcores=_NS)

    @functools.partial(
        pl.kernel,
        out_type=[
            jax.ShapeDtypeStruct((_NW * N,), jnp.float32),
            jax.ShapeDtypeStruct((_NW * N,), jnp.float32),
            jax.ShapeDtypeStruct((_NW * N,), jnp.float32),
        ],
        mesh=mesh,
        compiler_params=pltpu.CompilerParams(needs_layout_passes=False),
        scratch_types=[
            pltpu.VMEM((N,), jnp.float32),
            pltpu.VMEM((N,), jnp.float32),
            pltpu.VMEM((N,), jnp.float32),
            pltpu.VMEM((EPW,), jnp.int32),
            pltpu.VMEM((EPW,), jnp.int32),
            pltpu.VMEM((EPW,), jnp.float32),
            pltpu.VMEM((N,), jnp.float32),
            pltpu.VMEM((N,), jnp.float32),
            pltpu.VMEM((N,), jnp.float32),
        ],
    )
    def k(c_hbm, px_hbm, py_hbm, pz_hbm, row_hbm, col_hbm, zv_hbm,
          wpx_hbm, wpy_hbm, wpz_hbm,
          px_v, py_v, pz_v, idx_r, idx_c, c_all, awx, awy, awz):
        cid = lax.axis_index("c")
        sid = lax.axis_index("s")
        wid = cid * _NS + sid
        base0 = wid * EPW

        pltpu.sync_copy(row_hbm.at[pl.ds(base0, EPW)], idx_r)
        pltpu.sync_copy(col_hbm.at[pl.ds(base0, EPW)], idx_c)
        pltpu.sync_copy(c_hbm.at[pl.ds(base0, EPW)], c_all)
        pltpu.sync_copy(px_hbm, px_v)
        pltpu.sync_copy(py_hbm, py_v)
        pltpu.sync_copy(pz_hbm, pz_v)
        pltpu.sync_copy(zv_hbm, awx)
        pltpu.sync_copy(zv_hbm, awy)
        pltpu.sync_copy(zv_hbm, awz)

        def group(g, carry):
            sl = pl.ds(g * _L, _L)
            rv = idx_r[sl]
            cv = idx_c[sl]
            cc = c_all[sl]
            wx = (plsc.load_gather(px_v, [rv]) - plsc.load_gather(px_v, [cv])) * cc
            wy = (plsc.load_gather(py_v, [rv]) - plsc.load_gather(py_v, [cv])) * cc
            wz = (plsc.load_gather(pz_v, [rv]) - plsc.load_gather(pz_v, [cv])) * cc
            plsc.addupdate_scatter(awx, [cv], wx)
            plsc.addupdate_scatter(awy, [cv], wy)
            plsc.addupdate_scatter(awz, [cv], wz)
            return carry

        lax.fori_loop(0, NG, group, 0)

        pltpu.sync_copy(awx, wpx_hbm.at[pl.ds(wid * N, N)])
        pltpu.sync_copy(awy, wpy_hbm.at[pl.ds(wid * N, N)])
        pltpu.sync_copy(awz, wpz_hbm.at[pl.ds(wid * N, N)])

    return k(cvals, px, py, pz, row, col, zv)


def _tc_pos(wpx, wpy, wpz, wpx2, wpy2, wpz2):
    W, N = wpx.shape

    def body(x_ref, y_ref, z_ref, x2_ref, y2_ref, z2_ref,
             ox_ref, oy_ref, oz_ref):
        ox_ref[...] = (jnp.sum(x_ref[...], axis=0, keepdims=True)
                       + jnp.sum(x2_ref[...], axis=0, keepdims=True))
        oy_ref[...] = (jnp.sum(y_ref[...], axis=0, keepdims=True)
                       + jnp.sum(y2_ref[...], axis=0, keepdims=True))
        oz_ref[...] = (jnp.sum(z_ref[...], axis=0, keepdims=True)
                       + jnp.sum(z2_ref[...], axis=0, keepdims=True))

    return pl.pallas_call(
        body,
        grid=(1,),
        in_specs=[pl.BlockSpec((W, N), lambda i: (0, 0))] * 6,
        out_specs=[pl.BlockSpec((1, N), lambda i: (0, 0))] * 3,
        out_shape=[jax.ShapeDtypeStruct((1, N), jnp.float32)] * 3,
    )(wpx, wpy, wpz, wpx2, wpy2, wpz2)


def _tc_node(h, hmp, hmp2, Wh1, bh1, Wh2, bh2, gh, bgh):
    N, H = h.shape
    BN = 1000
    full = lambda i: (0, 0)
    col1 = lambda i: (i, 0)

    def body(h_ref, hmp_ref, hmp2_ref, w1_ref, b1_ref, w2_ref, b2_ref,
             g_ref, bg_ref, hu_ref):
        s = h_ref[...] + hmp_ref[0] + hmp_ref[1] + hmp2_ref[0] + hmp2_ref[1]
        x = jnp.dot(s, w1_ref[...], preferred_element_type=jnp.float32) + b1_ref[...]
        x = x * jax.nn.sigmoid(x)
        u = jnp.dot(x, w2_ref[...], preferred_element_type=jnp.float32) + b2_ref[...]
        mu = jnp.mean(u, axis=-1, keepdims=True)
        var = jnp.mean((u - mu) * (u - mu), axis=-1, keepdims=True)
        hu_ref[...] = (u - mu) / jnp.sqrt(var + 1e-5) * g_ref[...] + bg_ref[...]

    return pl.pallas_call(
        body,
        grid=(N // BN,),
        in_specs=[
            pl.BlockSpec((BN, H), col1),
            pl.BlockSpec((_NC, BN, H), lambda i: (0, i, 0)),
            pl.BlockSpec((_NC, BN, H), lambda i: (0, i, 0)),
            pl.BlockSpec((H, H), full),
            pl.BlockSpec((1, H), full),
            pl.BlockSpec((H, H), full),
            pl.BlockSpec((1, H), full),
            pl.BlockSpec((1, H), full),
            pl.BlockSpec((1, H), full),
        ],
        out_specs=[pl.BlockSpec((BN, H), col1)],
        out_shape=[jax.ShapeDtypeStruct((N, H), jnp.float32)],
    )(h, hmp, hmp2, Wh1, bh1, Wh2, bh2, gh, bgh)


def kernel(h, pos, edge_index, edge_attr, We1, be1, We2, be2, ge, bge,
           Wh1, bh1, Wh2, bh2, gh, bgh, Wx1, bx1, Wx2, bx2):
    N, H = h.shape
    E = edge_index.shape[1]

    row = jnp.clip(edge_index[0], 0, N - 1)
    col = jnp.clip(edge_index[1], 0, N - 1)
    px = pos[:, 0]
    py = pos[:, 1]
    pz = pos[:, 2]

    Wr = We1[:H]
    Wc = We1[H:2 * H]
    wd = We1[2 * H].reshape(1, H)

    Prow, Pcol = _tc_project(h, Wr, Wc)

    E0 = (E * 3) // 5
    zem = jnp.zeros((N, H), jnp.float32)
    zv = jnp.zeros((N,), jnp.float32)

    row0, row1 = row[:E0], row[E0:]
    col0, col1 = col[:E0], col[E0:]
    z0, d20 = _sc_gather(Prow, Pcol, px, py, pz, row0, col0)
    z1, d21 = _sc_gather(Prow, Pcol, px, py, pz, row1, col1)
    We2b = We2.astype(jnp.bfloat16)
    Wx1b = Wx1.astype(jnp.bfloat16)
    em0, c0 = _tc_edge(
        z0, d20.reshape(E0, 1), wd, be1.reshape(1, H), We2b,
        be2.reshape(1, H), ge.reshape(1, H), bge.reshape(1, H), Wx1b,
        bx1.reshape(1, H), Wx2, bx2.reshape(1, 1))
    em1, c1 = _tc_edge(
        z1, d21.reshape(E - E0, 1), wd, be1.reshape(1, H), We2b,
        be2.reshape(1, H), ge.reshape(1, H), bge.reshape(1, H), Wx1b,
        bx1.reshape(1, H), Wx2, bx2.reshape(1, 1))
    (hmp0,) = _sc_scatter_em(em0, col0, zem)
    (hmp1,) = _sc_scatter_em(em1, col1, zem)
    wpx0, wpy0, wpz0 = _sc_scatter_pos(
        c0.reshape(E0), px, py, pz, row0, col0, zv)
    wpx1, wpy1, wpz1 = _sc_scatter_pos(
        c1.reshape(E - E0), px, py, pz, row1, col1, zv)
    (hu,) = _tc_node(h, hmp0, hmp1, Wh1, bh1.reshape(1, H), Wh2,
                     bh2.reshape(1, H), gh.reshape(1, H), bgh.reshape(1, H))
    pux, puy, puz = _tc_pos(
        wpx0.reshape(_NW, N), wpy0.reshape(_NW, N), wpz0.reshape(_NW, N),
        wpx1.reshape(_NW, N), wpy1.reshape(_NW, N), wpz1.reshape(_NW, N))
    pos_updates = jnp.concatenate(
        [pux.reshape(N, 1), puy.reshape(N, 1), puz.reshape(N, 1)], axis=1)
    return hu, pos_updates
